# baseline (speedup 1.0000x reference)
import functools

import jax
import jax.numpy as jnp
from jax import lax
from jax.experimental import pallas as pl
from jax.experimental.pallas import tpu as pltpu
from jax.experimental.pallas import tpu_sc as plsc

OUT = 64
NC, NS = 2, 16
NW = NC * NS
SB = 640
GS = 128
NG = SB // GS
VL = 16
TROWS = 512


def _table_body(emb0_ref, emb1_ref, emb2_ref, w_ref, b_ref, out_ref):
    a = jnp.dot(emb0_ref[...], w_ref[0:64, :], preferred_element_type=jnp.float32)
    bb = jnp.dot(emb1_ref[...], w_ref[64:128, :], preferred_element_type=jnp.float32)
    c = jnp.dot(emb2_ref[...], w_ref[128:192, :], preferred_element_type=jnp.float32)
    r = lax.broadcasted_iota(jnp.int32, (TROWS, 1), 0)
    j32 = lax.broadcasted_iota(jnp.int32, (1, 32), 1)
    j8 = lax.broadcasted_iota(jnp.int32, (1, 8), 1)
    oh0 = ((r // 16) == j32).astype(jnp.float32)
    oh1 = (((r // 2) % 8) == j8).astype(jnp.float32)
    oh2 = ((r % 2) == j8).astype(jnp.float32)
    out_ref[...] = (
        jnp.dot(oh0, a, preferred_element_type=jnp.float32)
        + jnp.dot(oh1, bb, preferred_element_type=jnp.float32)
        + jnp.dot(oh2, c, preferred_element_type=jnp.float32)
        + b_ref[...]
    )


def _build_table(emb0, emb1, emb2, w, b):
    emb0p = jnp.zeros((32, OUT), jnp.float32).at[:emb0.shape[0]].set(emb0)
    emb1p = jnp.zeros((8, OUT), jnp.float32).at[:emb1.shape[0]].set(emb1)
    emb2p = jnp.zeros((8, OUT), jnp.float32).at[:emb2.shape[0]].set(emb2)
    return pl.pallas_call(
        _table_body,
        out_shape=jax.ShapeDtypeStruct((TROWS, OUT), jnp.float32),
    )(emb0p, emb1p, emb2p, w, b.reshape(1, OUT))


def _relayout_body(in_ref, out_ref):
    x = in_ref[...]
    n = x.shape[0]
    a = x[:, :OUT].reshape(n, 1, OUT)
    b = x[:, OUT:].reshape(n, 1, OUT)
    out_ref[...] = jnp.concatenate([a, b], axis=1).reshape(2 * n, OUT)


def _relayout(flat, epad):
    nsteps = epad // SB
    k = next(kk for kk in range(10, 0, -1) if nsteps % kk == 0)
    rb = SB * k
    return pl.pallas_call(
        _relayout_body,
        grid=(epad // rb,),
        in_specs=[pl.BlockSpec((rb // 2, 128), lambda i: (i, 0))],
        out_specs=pl.BlockSpec((rb, OUT), lambda i: (i, 0)),
        out_shape=jax.ShapeDtypeStruct((epad, OUT), jnp.float32),
    )(flat.reshape(epad // 2, 2 * OUT))


def _gather_body(nsteps, tbl_hbm, attr_hbm, out_hbm,
                 tbl_v, attr_v, idx_v, rows_v, tsem, csem, gsem, wsem):
    wid = lax.axis_index("s") * NC + lax.axis_index("c")
    n_w = (nsteps - wid + NW - 1) // NW

    @pl.when(lax.axis_index("s") == 0)
    def _():
        pltpu.async_copy(tbl_hbm, tbl_v, tsem).wait()

    plsc.subcore_barrier()

    def step(t, carry):
        j = wid + t * NW
        off = j * SB
        slot = lax.rem(t, 2)

        @pl.when(t == 0)
        def _():
            for c in range(3):
                pltpu.async_copy(
                    attr_hbm.at[c, pl.ds(off, SB)], attr_v.at[0, c], csem
                )

        for c in range(3):
            pltpu.make_async_copy(
                attr_hbm.at[c, pl.ds(off, SB)], attr_v.at[slot, c], csem
            ).wait()

        @pl.when(t + 1 < n_w)
        def _():
            off_n = (j + NW) * SB
            for c in range(3):
                pltpu.async_copy(
                    attr_hbm.at[c, pl.ds(off_n, SB)], attr_v.at[1 - slot, c], csem
                )

        def mk_idx(g, carry):
            sl = pl.ds(g * VL, VL)
            a0 = attr_v[slot, 0, sl]
            a1 = attr_v[slot, 1, sl]
            a2 = attr_v[slot, 2, sl]
            idx_v[slot, sl] = a0 * 16 + a1 * 2 + a2
            return carry

        lax.fori_loop(0, SB // VL, mk_idx, 0, unroll=8)

        for g in range(NG):
            pltpu.async_copy(
                tbl_v.at[idx_v.at[slot, pl.ds(g * GS, GS)]],
                rows_v.at[slot, pl.ds(g * GS, GS), :],
                gsem,
            )
        for g in range(NG):
            pltpu.make_async_copy(
                tbl_v.at[idx_v.at[slot, pl.ds(g * GS, GS)]],
                rows_v.at[slot, pl.ds(g * GS, GS), :],
                gsem,
            ).wait()

        @pl.when(t >= 1)
        def _():
            pltpu.make_async_copy(
                rows_v.at[slot, :, :], out_hbm.at[pl.ds(off, SB)], wsem
            ).wait()

        pltpu.async_copy(rows_v.at[slot, :, :], out_hbm.at[pl.ds(off, SB)], wsem)
        return carry

    lax.fori_loop(0, n_w, step, 0)

    @pl.when(n_w >= 1)
    def _():
        pltpu.make_async_copy(
            rows_v.at[0, :, :], out_hbm.at[pl.ds(0, SB)], wsem
        ).wait()


def kernel(edge_attr, emb0, emb1, emb2, W, b):
    e = edge_attr.shape[0]
    epad = ((e + SB - 1) // SB) * SB
    nsteps = epad // SB

    tbl = _build_table(emb0, emb1, emb2, W, b)

    attr_in = edge_attr.T
    if epad != e:
        attr_in = jnp.pad(attr_in, ((0, 0), (0, epad - e)))

    mesh = plsc.VectorSubcoreMesh(
        core_axis_name="c", subcore_axis_name="s", num_cores=NC, num_subcores=NS
    )
    out = pl.kernel(
        functools.partial(_gather_body, nsteps),
        out_type=jax.ShapeDtypeStruct((epad, OUT), jnp.float32),
        mesh=mesh,
        compiler_params=pltpu.CompilerParams(use_tc_tiling_on_sc=False),
        scratch_types=[
            pltpu.VMEM_SHARED((TROWS, OUT), jnp.float32),
            pltpu.VMEM((2, 3, SB), jnp.int32),
            pltpu.VMEM((2, SB), jnp.int32),
            pltpu.VMEM((2, SB, OUT), jnp.float32),
            pltpu.SemaphoreType.DMA,
            pltpu.SemaphoreType.DMA,
            pltpu.SemaphoreType.DMA,
            pltpu.SemaphoreType.DMA,
        ],
    )(tbl, attr_in)
    out = _relayout(out, epad)
    return out if epad == e else out[:e]


# speedup vs baseline: 1.3819x; 1.3819x over previous
import functools

import jax
import jax.numpy as jnp
from jax import lax
from jax.experimental import pallas as pl
from jax.experimental.pallas import tpu as pltpu
from jax.experimental.pallas import tpu_sc as plsc

OUT = 64
NC, NS = 2, 16
NW = NC * NS
SB = 640
GS = 128
NG = SB // GS
VL = 16
TROWS = 512


def _table_body(emb0_ref, emb1_ref, emb2_ref, w_ref, b_ref, out_ref):
    a = jnp.dot(emb0_ref[...], w_ref[0:64, :], preferred_element_type=jnp.float32)
    bb = jnp.dot(emb1_ref[...], w_ref[64:128, :], preferred_element_type=jnp.float32)
    c = jnp.dot(emb2_ref[...], w_ref[128:192, :], preferred_element_type=jnp.float32)
    r = lax.broadcasted_iota(jnp.int32, (TROWS, 1), 0)
    j32 = lax.broadcasted_iota(jnp.int32, (1, 32), 1)
    j8 = lax.broadcasted_iota(jnp.int32, (1, 8), 1)
    oh0 = ((r // 16) == j32).astype(jnp.float32)
    oh1 = (((r // 2) % 8) == j8).astype(jnp.float32)
    oh2 = ((r % 2) == j8).astype(jnp.float32)
    out_ref[...] = (
        jnp.dot(oh0, a, preferred_element_type=jnp.float32)
        + jnp.dot(oh1, bb, preferred_element_type=jnp.float32)
        + jnp.dot(oh2, c, preferred_element_type=jnp.float32)
        + b_ref[...]
    )


def _build_table(emb0, emb1, emb2, w, b):
    emb0p = jnp.zeros((32, OUT), jnp.float32).at[:emb0.shape[0]].set(emb0)
    emb1p = jnp.zeros((8, OUT), jnp.float32).at[:emb1.shape[0]].set(emb1)
    emb2p = jnp.zeros((8, OUT), jnp.float32).at[:emb2.shape[0]].set(emb2)
    return pl.pallas_call(
        _table_body,
        out_shape=jax.ShapeDtypeStruct((TROWS, OUT), jnp.float32),
    )(emb0p, emb1p, emb2p, w, b.reshape(1, OUT))


def _gather_body(nsteps, tbl_hbm, attr_hbm, out_hbm,
                 tbl_v, attr_v, idx_v, rows_v, tsem, csem, gsem, wsem):
    wid = lax.axis_index("s") * NC + lax.axis_index("c")
    n_w = (nsteps - wid + NW - 1) // NW

    @pl.when(lax.axis_index("s") == 0)
    def _():
        pltpu.async_copy(tbl_hbm, tbl_v, tsem).wait()

    plsc.subcore_barrier()

    def step(t, carry):
        j = wid + t * NW
        off = j * SB
        slot = lax.rem(t, 2)

        @pl.when(t == 0)
        def _():
            for c in range(3):
                pltpu.async_copy(
                    attr_hbm.at[c, pl.ds(off, SB)], attr_v.at[0, c], csem
                )

        for c in range(3):
            pltpu.make_async_copy(
                attr_hbm.at[c, pl.ds(off, SB)], attr_v.at[slot, c], csem
            ).wait()

        @pl.when(t + 1 < n_w)
        def _():
            off_n = (j + NW) * SB
            for c in range(3):
                pltpu.async_copy(
                    attr_hbm.at[c, pl.ds(off_n, SB)], attr_v.at[1 - slot, c], csem
                )

        def mk_idx(g, carry):
            sl = pl.ds(g * VL, VL)
            a0 = attr_v[slot, 0, sl]
            a1 = attr_v[slot, 1, sl]
            a2 = attr_v[slot, 2, sl]
            idx_v[slot, sl] = a0 * 16 + a1 * 2 + a2
            return carry

        lax.fori_loop(0, SB // VL, mk_idx, 0, unroll=8)

        for g in range(NG):
            pltpu.async_copy(
                tbl_v.at[idx_v.at[slot, pl.ds(g * GS, GS)]],
                rows_v.at[slot, pl.ds(g * GS, GS), :],
                gsem,
            )
        for g in range(NG):
            pltpu.make_async_copy(
                tbl_v.at[idx_v.at[slot, pl.ds(g * GS, GS)]],
                rows_v.at[slot, pl.ds(g * GS, GS), :],
                gsem,
            ).wait()

        @pl.when(t >= 1)
        def _():
            pltpu.make_async_copy(
                rows_v.at[slot, :, :], out_hbm.at[pl.ds(off, SB)], wsem
            ).wait()

        pltpu.async_copy(rows_v.at[slot, :, :], out_hbm.at[pl.ds(off, SB)], wsem)
        return carry

    lax.fori_loop(0, n_w, step, 0)

    @pl.when(n_w >= 1)
    def _():
        pltpu.make_async_copy(
            rows_v.at[0, :, :], out_hbm.at[pl.ds(0, SB)], wsem
        ).wait()


def kernel(edge_attr, emb0, emb1, emb2, W, b):
    e = edge_attr.shape[0]
    epad = ((e + SB - 1) // SB) * SB
    nsteps = epad // SB

    tbl = _build_table(emb0, emb1, emb2, W, b)

    attr_in = edge_attr.T
    if epad != e:
        attr_in = jnp.pad(attr_in, ((0, 0), (0, epad - e)))

    mesh = plsc.VectorSubcoreMesh(
        core_axis_name="c", subcore_axis_name="s", num_cores=NC, num_subcores=NS
    )
    out = pl.kernel(
        functools.partial(_gather_body, nsteps),
        out_type=jax.ShapeDtypeStruct((epad, OUT), jnp.float32),
        mesh=mesh,
        compiler_params=pltpu.CompilerParams(use_tc_tiling_on_sc=False),
        scratch_types=[
            pltpu.VMEM_SHARED((TROWS, OUT), jnp.float32),
            pltpu.VMEM((2, 3, SB), jnp.int32),
            pltpu.VMEM((2, SB), jnp.int32),
            pltpu.VMEM((2, SB, OUT), jnp.float32),
            pltpu.SemaphoreType.DMA,
            pltpu.SemaphoreType.DMA,
            pltpu.SemaphoreType.DMA,
            pltpu.SemaphoreType.DMA,
        ],
    )(tbl, attr_in)
    return out if epad == e else out[:e]
